# Initial kernel scaffold; baseline (speedup 1.0000x reference)
import jax
import jax.numpy as jnp
from jax.experimental import pallas as pl


def kernel(mm_data, dd_data, x_m, x_d, Wgx1, bgx1, Wgx2, bgx2, Wgy1, bgy1, Wgy2, bgy2, Wlx1, blx1, Wlx2, blx2, Wlx3, blx3, Wly1, bly1, Wly2, bly2, Wly3, bly3, mm_edge_index, dd_edge_index):
    raise NotImplementedError("write your pallas kernel here")


# speedup vs baseline: 4.7393x; 4.7393x over previous
import functools

import jax
import jax.numpy as jnp
from jax import lax
from jax.experimental import pallas as pl
from jax.experimental.pallas import tpu as pltpu
from jax.experimental.pallas import tpu_sc as plsc

N = 10000
FIN = 128
F1 = 256
F2 = 128
KOUT = 64
E = 320000
E_PAD = 327680
NC = 2
NS = 16

_mesh = plsc.VectorSubcoreMesh(
    core_axis_name="c", subcore_axis_name="s", num_cores=NC, num_subcores=NS)

_f32 = jnp.float32
_i32 = jnp.int32


PREP_CH = 2048
PREP_PER_W = E_PAD // (NC * NS)
PREP_CHUNKS = PREP_PER_W // PREP_CH


def _edge_prep_body(datam, datad, srcm2, dstm2, srcd2, dstd2, z1k,
                    wm_out, wd_out, degm_out, degd_out,
                    degm_s, degd_s,
                    srcix, dstix, ridx, dix1, wbuf, tmp1k, sem):
    c = lax.axis_index("c")
    s = lax.axis_index("s")
    wid = s * NC + c
    base = wid * PREP_PER_W

    @pl.when(s == 0)
    def _():
        pltpu.sync_copy(z1k, tmp1k)
        for t in range(10):
            pltpu.sync_copy(tmp1k, degm_s.at[pl.ds(1000 * t, 1000)])
            pltpu.sync_copy(tmp1k, degd_s.at[pl.ds(1000 * t, 1000)])
    plsc.subcore_barrier()

    for (dataf, src2, dst2, w_out, deg_s) in (
            (datam, srcm2, dstm2, wm_out, degm_s),
            (datad, srcd2, dstd2, wd_out, degd_s)):

        def _chunk(k, carry):
            eoff = base + k * PREP_CH
            goff = eoff // 128
            pltpu.sync_copy(src2.at[pl.ds(goff, 16)], srcix)
            pltpu.sync_copy(dst2.at[pl.ds(goff, 16)], dstix)

            def _vb(j, cy):
                r = j // 8
                cs = 16 * (j % 8)
                sv = srcix[r, 0, pl.ds(cs, 16)]
                dv = dstix[r, 0, pl.ds(cs, 16)]
                ridx[pl.ds(j * 16, 16)] = sv * N + dv
                dix1[pl.ds(j * 16, 16)] = dv
                return cy
            lax.fori_loop(0, 128, _vb, None)

            cps = [pltpu.async_copy(dataf.at[ridx.at[pl.ds(128 * g, 128)]],
                                    wbuf.at[pl.ds(128 * g, 128)], sem)
                   for g in range(16)]
            for cp in cps:
                cp.wait()

            pltpu.sync_copy(wbuf, w_out.at[pl.ds(eoff, PREP_CH)])
            for g in range(16):
                pltpu.sync_copy(wbuf.at[pl.ds(128 * g, 128)],
                                deg_s.at[dix1.at[pl.ds(128 * g, 128)]],
                                add=True)
            return carry
        lax.fori_loop(0, PREP_CHUNKS, _chunk, None)

    plsc.subcore_barrier()

    @pl.when(s < 10)
    def _():
        off = pl.multiple_of(1000 * s, 8)
        pltpu.sync_copy(degm_s.at[pl.ds(off, 1000)], tmp1k)
        pltpu.sync_copy(tmp1k, degm_out.at[c, s, 0])
        pltpu.sync_copy(degd_s.at[pl.ds(off, 1000)], tmp1k)
        pltpu.sync_copy(tmp1k, degd_out.at[c, s, 0])


_edge_prep = pl.kernel(
    _edge_prep_body,
    out_type=(jax.ShapeDtypeStruct((E_PAD,), _f32),
              jax.ShapeDtypeStruct((E_PAD,), _f32),
              jax.ShapeDtypeStruct((NC, 10, 1, 1000), _f32),
              jax.ShapeDtypeStruct((NC, 10, 1, 1000), _f32)),
    mesh=_mesh,
    scratch_types=[
        pltpu.VMEM_SHARED((N + 8,), _f32),
        pltpu.VMEM_SHARED((N + 8,), _f32),
        pltpu.VMEM((16, 1, 128), _i32),
        pltpu.VMEM((16, 1, 128), _i32),
        pltpu.VMEM((PREP_CH,), _i32),
        pltpu.VMEM((PREP_CH,), _i32),
        pltpu.VMEM((PREP_CH,), _f32),
        pltpu.VMEM((1000,), _f32),
        pltpu.SemaphoreType.DMA,
    ],
)


AGG_CH = 256
AGG_SCH = 1024
AGG_PER_T = E_PAD // NS
AGG_CHUNKS = AGG_PER_T // AGG_SCH


def _aggregate_body(fc, esplit, hm2, hd2, wm2, wd2, srcm2, dstm2, srcd2,
                    dstd2, zfc, aggm_out, aggd_out,
                    acc_s, srcix, dstix, gidx, dix1, wv, rows, sem):
    c = lax.axis_index("c")
    s = lax.axis_index("s")
    nq = fc // 16
    per_t = E_PAD // (NC * NS) if esplit else E_PAD // NS
    nchunks = per_t // AGG_SCH

    for (h2, w2, src2, dst2, agg_out) in (
            (hm2, wm2, srcm2, dstm2, aggm_out),
            (hd2, wd2, srcd2, dstd2, aggd_out)):

        @pl.when(s < 10)
        def _():
            pltpu.sync_copy(zfc, rows.at[pl.ds(0, 200)])
            for t in range(5):
                off = pl.multiple_of(1000 * s + 200 * t, 8)
                pltpu.sync_copy(rows.at[pl.ds(0, 200)], acc_s.at[pl.ds(off, 200)])
        plsc.subcore_barrier()

        def _chunk(k, cy):
            eoff = s * per_t + k * AGG_SCH
            if esplit:
                eoff = eoff + c * (E_PAD // NC)
            goff = eoff // 128
            pltpu.sync_copy(src2.at[pl.ds(goff, 8)], srcix)
            pltpu.sync_copy(dst2.at[pl.ds(goff, 8)], dstix)
            woff = pl.multiple_of(eoff // 16, 8)
            pltpu.sync_copy(w2.at[pl.ds(woff, AGG_SCH // 16)], wv)

            def _vb(j, cz):
                r = j // 8
                cs = 16 * (j % 8)
                sv = srcix[r, 0, pl.ds(cs, 16)]
                gidx[pl.ds(j * 16, 16)] = sv if esplit else sv * 2 + c
                dix1[pl.ds(j * 16, 16)] = dstix[r, 0, pl.ds(cs, 16)]
                return cz
            lax.fori_loop(0, 64, _vb, None)

            for h in range(4):
                cps = [pltpu.async_copy(
                           h2.at[gidx.at[pl.ds(256 * h + 128 * g, 128)]],
                           rows.at[pl.ds(128 * g, 128)], sem)
                       for g in range(2)]
                for cp in cps:
                    cp.wait()

                def _mul(jj, cz):
                    wg = wv[h * 16 + jj, pl.ds(0, 16)]
                    for lane in range(16):
                        e = jj * 16 + lane
                        ws = wg[lane]
                        for q in range(nq):
                            rows[e, pl.ds(16 * q, 16)] = (
                                rows[e, pl.ds(16 * q, 16)] * ws)
                    return cz
                lax.fori_loop(0, AGG_CH // 16, _mul, None)

                for g in range(2):
                    pltpu.sync_copy(
                        rows.at[pl.ds(128 * g, 128)],
                        acc_s.at[dix1.at[pl.ds(256 * h + 128 * g, 128)]],
                        add=True)
            return cy
        lax.fori_loop(0, nchunks, _chunk, None)
        plsc.subcore_barrier()

        @pl.when(s < 10)
        def _():
            for t in range(5):
                off = pl.multiple_of(1000 * s + 200 * t, 8)
                pltpu.sync_copy(acc_s.at[pl.ds(off, 200)], rows.at[pl.ds(0, 200)])
                pltpu.sync_copy(rows.at[pl.ds(0, 200)],
                                agg_out.at[c].at[pl.ds(off, 200)])
        plsc.subcore_barrier()


def _make_aggregate(f, esplit):
    fc = f if esplit else f // 2
    return pl.kernel(
        functools.partial(_aggregate_body, fc, esplit),
        out_type=(jax.ShapeDtypeStruct((NC, N, fc), _f32),
                  jax.ShapeDtypeStruct((NC, N, fc), _f32)),
        mesh=_mesh,
        scratch_types=[
            pltpu.VMEM_SHARED((N + 8, fc), _f32),
            pltpu.VMEM((8, 1, 128), _i32),
            pltpu.VMEM((8, 1, 128), _i32),
            pltpu.VMEM((AGG_SCH,), _i32),
            pltpu.VMEM((AGG_SCH,), _i32),
            pltpu.VMEM((AGG_SCH // 16, 16), _f32),
            pltpu.VMEM((AGG_CH, fc), _f32),
            pltpu.SemaphoreType.DMA,
        ],
    )


_aggregate_f1 = _make_aggregate(F1, False)
_aggregate_f2 = _make_aggregate(F2, True)


_RB = 1000


def _dinv_body(degm_ref, degd_ref, dm_ref, dd_ref):
    dm_ref[...] = lax.rsqrt(degm_ref[0, :] + degm_ref[1, :] + 1.0)
    dd_ref[...] = lax.rsqrt(degd_ref[0, :] + degd_ref[1, :] + 1.0)


_dinv_call = pl.pallas_call(
    _dinv_body,
    out_shape=(jax.ShapeDtypeStruct((N,), _f32),
               jax.ShapeDtypeStruct((N,), _f32)),
)


def _xw_body(x_ref, w_ref, dinv_ref, out_ref):
    out_ref[...] = dinv_ref[...] * jnp.dot(
        x_ref[...], w_ref[...], preferred_element_type=_f32)


def _xw_call(f_in, f_out):
    return pl.pallas_call(
        _xw_body,
        grid=(N // _RB,),
        in_specs=[
            pl.BlockSpec((_RB, f_in), lambda i: (i, 0)),
            pl.BlockSpec((f_in, f_out), lambda i: (0, 0)),
            pl.BlockSpec((_RB, 1), lambda i: (i, 0)),
        ],
        out_specs=pl.BlockSpec((_RB, f_out), lambda i: (i, 0)),
        out_shape=jax.ShapeDtypeStruct((N, f_out), _f32),
    )


def _layer2_body(agg_ref, ht_ref, dinv_ref, b1_ref, w2_ref, out_ref):
    aggf = jnp.concatenate([agg_ref[0], agg_ref[1]], axis=-1)
    z = jnp.maximum(dinv_ref[...] * (aggf + ht_ref[...]) + b1_ref[...], 0.0)
    out_ref[...] = dinv_ref[...] * jnp.dot(
        z, w2_ref[...], preferred_element_type=_f32)


_layer2_call = pl.pallas_call(
    _layer2_body,
    grid=(N // _RB,),
    in_specs=[
        pl.BlockSpec((NC, _RB, F1 // 2), lambda i: (0, i, 0)),
        pl.BlockSpec((_RB, F1), lambda i: (i, 0)),
        pl.BlockSpec((_RB, 1), lambda i: (i, 0)),
        pl.BlockSpec((1, F1), lambda i: (0, 0)),
        pl.BlockSpec((F1, F2), lambda i: (0, 0)),
    ],
    out_specs=pl.BlockSpec((_RB, F2), lambda i: (i, 0)),
    out_shape=jax.ShapeDtypeStruct((N, F2), _f32),
)


def _head_body(agg_ref, ht_ref, dinv_ref, b2_ref,
               wl1_ref, bl1_ref, wl2_ref, bl2_ref, wl3_ref, bl3_ref, out_ref):
    aggf = agg_ref[0] + agg_ref[1]
    xx = jnp.maximum(dinv_ref[...] * (aggf + ht_ref[...]) + b2_ref[...], 0.0)
    x1 = jnp.maximum(jnp.dot(xx, wl1_ref[...], preferred_element_type=_f32)
                     + bl1_ref[...], 0.0)
    x2 = jnp.maximum(jnp.dot(x1, wl2_ref[...], preferred_element_type=_f32)
                     + bl2_ref[...], 0.0)
    out_ref[...] = jnp.maximum(
        jnp.dot(x2, wl3_ref[...], preferred_element_type=_f32)
        + bl3_ref[...], 0.0)


_head_call = pl.pallas_call(
    _head_body,
    grid=(N // _RB,),
    in_specs=[
        pl.BlockSpec((NC, _RB, F2), lambda i: (0, i, 0)),
        pl.BlockSpec((_RB, F2), lambda i: (i, 0)),
        pl.BlockSpec((_RB, 1), lambda i: (i, 0)),
        pl.BlockSpec((1, F2), lambda i: (0, 0)),
        pl.BlockSpec((F2, 256), lambda i: (0, 0)),
        pl.BlockSpec((1, 256), lambda i: (0, 0)),
        pl.BlockSpec((256, 128), lambda i: (0, 0)),
        pl.BlockSpec((1, 128), lambda i: (0, 0)),
        pl.BlockSpec((128, KOUT), lambda i: (0, 0)),
        pl.BlockSpec((1, KOUT), lambda i: (0, 0)),
    ],
    out_specs=pl.BlockSpec((_RB, KOUT), lambda i: (i, 0)),
    out_shape=jax.ShapeDtypeStruct((N, KOUT), _f32),
)

_FRB = 400


def _final_body(xf_ref, yf_ref, out_ref):
    out_ref[...] = lax.dot_general(
        xf_ref[...], yf_ref[...], (((1,), (1,)), ((), ())),
        preferred_element_type=_f32)


_final_call = pl.pallas_call(
    _final_body,
    grid=(N // _FRB,),
    in_specs=[
        pl.BlockSpec((_FRB, KOUT), lambda i: (i, 0)),
        pl.BlockSpec((N, KOUT), lambda i: (0, 0)),
    ],
    out_specs=pl.BlockSpec((_FRB, N), lambda i: (i, 0)),
    out_shape=jax.ShapeDtypeStruct((N, N), _f32),
)


def _pad_edges(ei):
    ei = ei.astype(_i32)
    src = jnp.concatenate([ei[0], jnp.zeros((E_PAD - E,), _i32)])
    dst = jnp.concatenate([ei[1], jnp.full((E_PAD - E,), N, _i32)])
    return (src.reshape(E_PAD // 128, 1, 128),
            dst.reshape(E_PAD // 128, 1, 128))


def kernel(mm_data, dd_data, x_m, x_d, Wgx1, bgx1, Wgx2, bgx2, Wgy1, bgy1,
           Wgy2, bgy2, Wlx1, blx1, Wlx2, blx2, Wlx3, blx3, Wly1, bly1,
           Wly2, bly2, Wly3, bly3, mm_edge_index, dd_edge_index):
    srcm2, dstm2 = _pad_edges(mm_edge_index)
    srcd2, dstd2 = _pad_edges(dd_edge_index)
    datamf = mm_data.reshape(N * N)
    datadf = dd_data.reshape(N * N)
    z1k = jnp.zeros((1000,), _f32)
    z200a = jnp.zeros((200, F1 // 2), _f32)

    wm, wd, degm, degd = _edge_prep(datamf, datadf,
                                    srcm2, dstm2, srcd2, dstd2, z1k)
    wm2 = wm.reshape(E_PAD // 16, 16)
    wd2 = wd.reshape(E_PAD // 16, 16)

    dinv_m, dinv_d = _dinv_call(degm.reshape(NC, N), degd.reshape(NC, N))
    dm2 = dinv_m.reshape(N, 1)
    dd2 = dinv_d.reshape(N, 1)

    xw = _xw_call(FIN, F1)
    htm1 = xw(x_m, Wgx1, dm2)
    htd1 = xw(x_d, Wgy1, dd2)

    aggm1, aggd1 = _aggregate_f1(htm1.reshape(2 * N, F1 // 2),
                                 htd1.reshape(2 * N, F1 // 2),
                                 wm2, wd2, srcm2, dstm2, srcd2, dstd2, z200a)

    b1m = bgx1.reshape(1, F1)
    b1d = bgy1.reshape(1, F1)
    htm2 = _layer2_call(aggm1, htm1, dm2, b1m, Wgx2)
    htd2 = _layer2_call(aggd1, htd1, dd2, b1d, Wgy2)

    aggm2, aggd2 = _aggregate_f2(htm2, htd2,
                                 wm2, wd2, srcm2, dstm2, srcd2, dstd2, z200a)

    xf = _head_call(aggm2, htm2, dm2, bgx2.reshape(1, F2),
                    Wlx1, blx1.reshape(1, 256), Wlx2, blx2.reshape(1, 128),
                    Wlx3, blx3.reshape(1, KOUT))
    yf = _head_call(aggd2, htd2, dd2, bgy2.reshape(1, F2),
                    Wly1, bly1.reshape(1, 256), Wly2, bly2.reshape(1, 128),
                    Wly3, bly3.reshape(1, KOUT))

    return _final_call(xf, yf)


# speedup vs baseline: 5.0548x; 1.0666x over previous
import functools

import jax
import jax.numpy as jnp
from jax import lax
from jax.experimental import pallas as pl
from jax.experimental.pallas import tpu as pltpu
from jax.experimental.pallas import tpu_sc as plsc

N = 10000
FIN = 128
F1 = 256
F2 = 128
KOUT = 64
E = 320000
E_PAD = 327680
NC = 2
NS = 16

_mesh = plsc.VectorSubcoreMesh(
    core_axis_name="c", subcore_axis_name="s", num_cores=NC, num_subcores=NS)

_f32 = jnp.float32
_i32 = jnp.int32


PREP_CH = 2048
PREP_PER_W = E_PAD // (NC * NS)
PREP_CHUNKS = PREP_PER_W // PREP_CH


def _edge_prep_body(datam, datad, fidxm, fidxd, dstm1, dstd1, z1k,
                    wm_out, wd_out, degm_out, degd_out,
                    degm_s, degd_s,
                    ridx, dix1, wbuf, tmp1k, sem, semi):
    c = lax.axis_index("c")
    s = lax.axis_index("s")
    wid = s * NC + c
    base = wid * PREP_PER_W

    @pl.when(s == 0)
    def _():
        pltpu.sync_copy(z1k, tmp1k)
        for t in range(10):
            pltpu.sync_copy(tmp1k, degm_s.at[pl.ds(1000 * t, 1000)])
            pltpu.sync_copy(tmp1k, degd_s.at[pl.ds(1000 * t, 1000)])
    plsc.subcore_barrier()

    for (dataf, fidx, dst1, w_out, deg_s) in (
            (datam, fidxm, dstm1, wm_out, degm_s),
            (datad, fidxd, dstd1, wd_out, degd_s)):

        def _chunk(k, carry):
            eoff = base + k * PREP_CH
            c1 = pltpu.async_copy(fidx.at[pl.ds(eoff, PREP_CH)], ridx, semi)
            c2 = pltpu.async_copy(dst1.at[pl.ds(eoff, PREP_CH)], dix1, semi)
            c1.wait()
            c2.wait()

            cps = [pltpu.async_copy(dataf.at[ridx.at[pl.ds(128 * g, 128)]],
                                    wbuf.at[pl.ds(128 * g, 128)], sem)
                   for g in range(16)]
            for cp in cps:
                cp.wait()

            pltpu.sync_copy(wbuf, w_out.at[pl.ds(eoff, PREP_CH)])
            for g in range(16):
                pltpu.sync_copy(wbuf.at[pl.ds(128 * g, 128)],
                                deg_s.at[dix1.at[pl.ds(128 * g, 128)]],
                                add=True)
            return carry
        lax.fori_loop(0, PREP_CHUNKS, _chunk, None)

    plsc.subcore_barrier()

    @pl.when(s < 10)
    def _():
        off = pl.multiple_of(1000 * s, 8)
        pltpu.sync_copy(degm_s.at[pl.ds(off, 1000)], tmp1k)
        pltpu.sync_copy(tmp1k, degm_out.at[c, s, 0])
        pltpu.sync_copy(degd_s.at[pl.ds(off, 1000)], tmp1k)
        pltpu.sync_copy(tmp1k, degd_out.at[c, s, 0])


_edge_prep = pl.kernel(
    _edge_prep_body,
    out_type=(jax.ShapeDtypeStruct((E_PAD,), _f32),
              jax.ShapeDtypeStruct((E_PAD,), _f32),
              jax.ShapeDtypeStruct((NC, 10, 1, 1000), _f32),
              jax.ShapeDtypeStruct((NC, 10, 1, 1000), _f32)),
    mesh=_mesh,
    scratch_types=[
        pltpu.VMEM_SHARED((N + 8,), _f32),
        pltpu.VMEM_SHARED((N + 8,), _f32),
        pltpu.VMEM((PREP_CH,), _i32),
        pltpu.VMEM((PREP_CH,), _i32),
        pltpu.VMEM((PREP_CH,), _f32),
        pltpu.VMEM((1000,), _f32),
        pltpu.SemaphoreType.DMA,
        pltpu.SemaphoreType.DMA,
    ],
)


AGG_SCH = 1024


def _aggregate_body(fc, esplit, hm2, hd2, wm2, wd2, gm0, gm1, gd0, gd1,
                    dstm1, dstd1, zfc, aggm_out, aggd_out,
                    acc_s, gidx, dix1, wv, rows,
                    semi, semg0, semg1, sems0, sems1):
    c = lax.axis_index("c")
    s = lax.axis_index("s")
    nq = fc // 16
    per_t = E_PAD // (NC * NS) if esplit else E_PAD // NS
    nsch = per_t // AGG_SCH
    semg = (semg0, semg1)
    sems = (sems0, sems1)

    for (h2, w2, g0a, g1a, dst1, agg_out) in (
            (hm2, wm2, gm0, gm1, dstm1, aggm_out),
            (hd2, wd2, gd0, gd1, dstd1, aggd_out)):

        @pl.when(s < 10)
        def _():
            pltpu.sync_copy(zfc, rows.at[pl.ds(0, 200)])
            for t in range(5):
                off = pl.multiple_of(1000 * s + 200 * t, 8)
                pltpu.sync_copy(rows.at[pl.ds(0, 200)],
                                acc_s.at[pl.ds(off, 200)])
        plsc.subcore_barrier()

        tbase = s * per_t
        if esplit:
            tbase = tbase + c * (E_PAD // NC)

        def _load_idx(eoff):
            @pl.when(c == 0)
            def _():
                pltpu.async_copy(g0a.at[pl.ds(eoff, AGG_SCH)], gidx, semi)

            @pl.when(c == 1)
            def _():
                pltpu.async_copy(g1a.at[pl.ds(eoff, AGG_SCH)], gidx, semi)
            c2 = pltpu.async_copy(dst1.at[pl.ds(eoff, AGG_SCH)], dix1, semi)
            woff = pl.multiple_of(eoff // 16, 8)
            c3 = pltpu.async_copy(w2.at[pl.ds(woff, AGG_SCH // 16)], wv, semi)
            pltpu.make_async_copy(dst1.at[pl.ds(eoff, AGG_SCH)], gidx,
                                  semi).wait()
            c2.wait()
            c3.wait()

        def _issue_gather(gg, sl):
            pltpu.async_copy(h2.at[gidx.at[pl.ds(128 * gg, 128)]],
                             rows.at[pl.ds(128 * sl, 128)], semg[sl])

        def _wait_gather(gg, sl):
            pltpu.make_async_copy(h2.at[gidx.at[pl.ds(128 * gg, 128)]],
                                  rows.at[pl.ds(128 * sl, 128)],
                                  semg[sl]).wait()

        def _issue_scatter(gg, sl):
            pltpu.async_copy(rows.at[pl.ds(128 * sl, 128)],
                             acc_s.at[dix1.at[pl.ds(128 * gg, 128)]],
                             sems[sl], add=True)

        def _wait_scatter(gg, sl):
            pltpu.make_async_copy(rows.at[pl.ds(128 * sl, 128)],
                                  acc_s.at[dix1.at[pl.ds(128 * gg, 128)]],
                                  sems[sl]).wait()

        def _mul_group(gg, sl):
            def _mul(jj, cz):
                wg = wv[8 * gg + jj, pl.ds(0, 16)]
                for lane in range(16):
                    e = 128 * sl + jj * 16 + lane
                    ws = wg[lane]
                    for q in range(nq):
                        rows[e, pl.ds(16 * q, 16)] = (
                            rows[e, pl.ds(16 * q, 16)] * ws)
                return cz
            lax.fori_loop(0, 8, _mul, None)

        _load_idx(tbase)
        _issue_gather(0, 0)
        _issue_gather(1, 1)

        def _chunk(k, cy):
            for gg in range(8):
                sl = gg & 1
                _wait_gather(gg, sl)
                _mul_group(gg, sl)
                _issue_scatter(gg, sl)
                if gg < 6:
                    _wait_scatter(gg, sl)
                    _issue_gather(gg + 2, sl)
            _wait_scatter(6, 0)
            _wait_scatter(7, 1)

            @pl.when(k + 1 < nsch)
            def _():
                _load_idx(tbase + (k + 1) * AGG_SCH)
                _issue_gather(0, 0)
                _issue_gather(1, 1)
            return cy
        lax.fori_loop(0, nsch, _chunk, None)
        plsc.subcore_barrier()

        @pl.when(s < 10)
        def _():
            for t in range(5):
                off = pl.multiple_of(1000 * s + 200 * t, 8)
                pltpu.sync_copy(acc_s.at[pl.ds(off, 200)],
                                rows.at[pl.ds(0, 200)])
                pltpu.sync_copy(rows.at[pl.ds(0, 200)],
                                agg_out.at[c].at[pl.ds(off, 200)])
        plsc.subcore_barrier()


def _make_aggregate(f, esplit):
    fc = f if esplit else f // 2
    return pl.kernel(
        functools.partial(_aggregate_body, fc, esplit),
        out_type=(jax.ShapeDtypeStruct((NC, N, fc), _f32),
                  jax.ShapeDtypeStruct((NC, N, fc), _f32)),
        mesh=_mesh,
        scratch_types=[
            pltpu.VMEM_SHARED((N + 1, fc), _f32),
            pltpu.VMEM((AGG_SCH,), _i32),
            pltpu.VMEM((AGG_SCH,), _i32),
            pltpu.VMEM((AGG_SCH // 16, 16), _f32),
            pltpu.VMEM((256, fc), _f32),
            pltpu.SemaphoreType.DMA,
            pltpu.SemaphoreType.DMA,
            pltpu.SemaphoreType.DMA,
            pltpu.SemaphoreType.DMA,
            pltpu.SemaphoreType.DMA,
        ],
    )


_aggregate_f1 = _make_aggregate(F1, False)
_aggregate_f2 = _make_aggregate(F2, True)


_RB = 1000


def _dinv_body(degm_ref, degd_ref, dm_ref, dd_ref):
    dm_ref[...] = lax.rsqrt(degm_ref[0, :] + degm_ref[1, :] + 1.0)
    dd_ref[...] = lax.rsqrt(degd_ref[0, :] + degd_ref[1, :] + 1.0)


_dinv_call = pl.pallas_call(
    _dinv_body,
    out_shape=(jax.ShapeDtypeStruct((N,), _f32),
               jax.ShapeDtypeStruct((N,), _f32)),
)


def _xw_body(x_ref, w_ref, dinv_ref, out_ref):
    out_ref[...] = dinv_ref[...] * jnp.dot(
        x_ref[...], w_ref[...], preferred_element_type=_f32)


def _xw_call(f_in, f_out):
    return pl.pallas_call(
        _xw_body,
        grid=(N // _RB,),
        in_specs=[
            pl.BlockSpec((_RB, f_in), lambda i: (i, 0)),
            pl.BlockSpec((f_in, f_out), lambda i: (0, 0)),
            pl.BlockSpec((_RB, 1), lambda i: (i, 0)),
        ],
        out_specs=pl.BlockSpec((_RB, f_out), lambda i: (i, 0)),
        out_shape=jax.ShapeDtypeStruct((N, f_out), _f32),
    )


def _layer2_body(agg_ref, ht_ref, dinv_ref, b1_ref, w2_ref, out_ref):
    aggf = jnp.concatenate([agg_ref[0], agg_ref[1]], axis=-1)
    z = jnp.maximum(dinv_ref[...] * (aggf + ht_ref[...]) + b1_ref[...], 0.0)
    out_ref[...] = dinv_ref[...] * jnp.dot(
        z, w2_ref[...], preferred_element_type=_f32)


_layer2_call = pl.pallas_call(
    _layer2_body,
    grid=(N // _RB,),
    in_specs=[
        pl.BlockSpec((NC, _RB, F1 // 2), lambda i: (0, i, 0)),
        pl.BlockSpec((_RB, F1), lambda i: (i, 0)),
        pl.BlockSpec((_RB, 1), lambda i: (i, 0)),
        pl.BlockSpec((1, F1), lambda i: (0, 0)),
        pl.BlockSpec((F1, F2), lambda i: (0, 0)),
    ],
    out_specs=pl.BlockSpec((_RB, F2), lambda i: (i, 0)),
    out_shape=jax.ShapeDtypeStruct((N, F2), _f32),
)


def _head_body(agg_ref, ht_ref, dinv_ref, b2_ref,
               wl1_ref, bl1_ref, wl2_ref, bl2_ref, wl3_ref, bl3_ref, out_ref):
    aggf = agg_ref[0] + agg_ref[1]
    xx = jnp.maximum(dinv_ref[...] * (aggf + ht_ref[...]) + b2_ref[...], 0.0)
    x1 = jnp.maximum(jnp.dot(xx, wl1_ref[...], preferred_element_type=_f32)
                     + bl1_ref[...], 0.0)
    x2 = jnp.maximum(jnp.dot(x1, wl2_ref[...], preferred_element_type=_f32)
                     + bl2_ref[...], 0.0)
    out_ref[...] = jnp.maximum(
        jnp.dot(x2, wl3_ref[...], preferred_element_type=_f32)
        + bl3_ref[...], 0.0)


_head_call = pl.pallas_call(
    _head_body,
    grid=(N // _RB,),
    in_specs=[
        pl.BlockSpec((NC, _RB, F2), lambda i: (0, i, 0)),
        pl.BlockSpec((_RB, F2), lambda i: (i, 0)),
        pl.BlockSpec((_RB, 1), lambda i: (i, 0)),
        pl.BlockSpec((1, F2), lambda i: (0, 0)),
        pl.BlockSpec((F2, 256), lambda i: (0, 0)),
        pl.BlockSpec((1, 256), lambda i: (0, 0)),
        pl.BlockSpec((256, 128), lambda i: (0, 0)),
        pl.BlockSpec((1, 128), lambda i: (0, 0)),
        pl.BlockSpec((128, KOUT), lambda i: (0, 0)),
        pl.BlockSpec((1, KOUT), lambda i: (0, 0)),
    ],
    out_specs=pl.BlockSpec((_RB, KOUT), lambda i: (i, 0)),
    out_shape=jax.ShapeDtypeStruct((N, KOUT), _f32),
)

_FRB = 400


def _final_body(xf_ref, yf_ref, out_ref):
    out_ref[...] = lax.dot_general(
        xf_ref[...], yf_ref[...], (((1,), (1,)), ((), ())),
        preferred_element_type=_f32)


_final_call = pl.pallas_call(
    _final_body,
    grid=(N // _FRB,),
    in_specs=[
        pl.BlockSpec((_FRB, KOUT), lambda i: (i, 0)),
        pl.BlockSpec((N, KOUT), lambda i: (0, 0)),
    ],
    out_specs=pl.BlockSpec((_FRB, N), lambda i: (i, 0)),
    out_shape=jax.ShapeDtypeStruct((N, N), _f32),
)


def _pad_edges(ei):
    ei = ei.astype(_i32)
    src = jnp.concatenate([ei[0], jnp.zeros((E_PAD - E,), _i32)])
    dst = jnp.concatenate([ei[1], jnp.full((E_PAD - E,), N, _i32)])
    return src, dst


def kernel(mm_data, dd_data, x_m, x_d, Wgx1, bgx1, Wgx2, bgx2, Wgy1, bgy1,
           Wgy2, bgy2, Wlx1, blx1, Wlx2, blx2, Wlx3, blx3, Wly1, bly1,
           Wly2, bly2, Wly3, bly3, mm_edge_index, dd_edge_index):
    srcm1, dstm1 = _pad_edges(mm_edge_index)
    srcd1, dstd1 = _pad_edges(dd_edge_index)
    fidxm = srcm1 * N + dstm1
    fidxd = srcd1 * N + dstd1
    gm0 = srcm1 * 2
    gm1 = gm0 + 1
    gd0 = srcd1 * 2
    gd1 = gd0 + 1
    datamf = mm_data.reshape(N * N)
    datadf = dd_data.reshape(N * N)
    z1k = jnp.zeros((1000,), _f32)
    z100 = jnp.zeros((200, F1 // 2), _f32)

    wm, wd, degm, degd = _edge_prep(datamf, datadf, fidxm, fidxd,
                                    dstm1, dstd1, z1k)
    wm2 = wm.reshape(E_PAD // 16, 16)
    wd2 = wd.reshape(E_PAD // 16, 16)

    dinv_m, dinv_d = _dinv_call(degm.reshape(NC, N), degd.reshape(NC, N))
    dm2 = dinv_m.reshape(N, 1)
    dd2 = dinv_d.reshape(N, 1)

    xw = _xw_call(FIN, F1)
    htm1 = xw(x_m, Wgx1, dm2)
    htd1 = xw(x_d, Wgy1, dd2)

    aggm1, aggd1 = _aggregate_f1(htm1.reshape(2 * N, F1 // 2),
                                 htd1.reshape(2 * N, F1 // 2),
                                 wm2, wd2, gm0, gm1, gd0, gd1,
                                 dstm1, dstd1, z100)

    b1m = bgx1.reshape(1, F1)
    b1d = bgy1.reshape(1, F1)
    htm2 = _layer2_call(aggm1, htm1, dm2, b1m, Wgx2)
    htd2 = _layer2_call(aggd1, htd1, dd2, b1d, Wgy2)

    aggm2, aggd2 = _aggregate_f2(htm2, htd2,
                                 wm2, wd2, srcm1, srcm1, srcd1, srcd1,
                                 dstm1, dstd1, z100)

    xf = _head_call(aggm2, htm2, dm2, bgx2.reshape(1, F2),
                    Wlx1, blx1.reshape(1, 256), Wlx2, blx2.reshape(1, 128),
                    Wlx3, blx3.reshape(1, KOUT))
    yf = _head_call(aggd2, htd2, dd2, bgy2.reshape(1, F2),
                    Wly1, bly1.reshape(1, 256), Wly2, bly2.reshape(1, 128),
                    Wly3, bly3.reshape(1, KOUT))

    return _final_call(xf, yf)


# speedup vs baseline: 5.0639x; 1.0018x over previous
import functools

import jax
import jax.numpy as jnp
from jax import lax
from jax.experimental import pallas as pl
from jax.experimental.pallas import tpu as pltpu
from jax.experimental.pallas import tpu_sc as plsc

N = 10000
FIN = 128
F1 = 256
F2 = 128
KOUT = 64
E = 320000
E_PAD = 327680
NC = 2
NS = 16

_mesh = plsc.VectorSubcoreMesh(
    core_axis_name="c", subcore_axis_name="s", num_cores=NC, num_subcores=NS)

_f32 = jnp.float32
_i32 = jnp.int32


PREP_CH = 2048
PREP_PER_W = E_PAD // (NC * NS)
PREP_CHUNKS = PREP_PER_W // PREP_CH


def _edge_prep_body(datam, datad, fidxm, fidxd, dstm1, dstd1, z1k,
                    wm_out, wd_out, degm_out, degd_out,
                    degm_s, degd_s,
                    ridx, dix1, wbuf, tmp1k, sem, semi):
    c = lax.axis_index("c")
    s = lax.axis_index("s")
    wid = s * NC + c
    base = wid * PREP_PER_W

    @pl.when(s == 0)
    def _():
        pltpu.sync_copy(z1k, tmp1k)
        for t in range(10):
            pltpu.sync_copy(tmp1k, degm_s.at[pl.ds(1000 * t, 1000)])
            pltpu.sync_copy(tmp1k, degd_s.at[pl.ds(1000 * t, 1000)])
    plsc.subcore_barrier()

    for (dataf, fidx, dst1, w_out, deg_s) in (
            (datam, fidxm, dstm1, wm_out, degm_s),
            (datad, fidxd, dstd1, wd_out, degd_s)):

        def _chunk(k, carry):
            eoff = base + k * PREP_CH
            c1 = pltpu.async_copy(fidx.at[pl.ds(eoff, PREP_CH)], ridx, semi)
            c2 = pltpu.async_copy(dst1.at[pl.ds(eoff, PREP_CH)], dix1, semi)
            c1.wait()
            c2.wait()

            cps = [pltpu.async_copy(dataf.at[ridx.at[pl.ds(128 * g, 128)]],
                                    wbuf.at[pl.ds(128 * g, 128)], sem)
                   for g in range(16)]
            for cp in cps:
                cp.wait()

            pltpu.sync_copy(wbuf, w_out.at[pl.ds(eoff, PREP_CH)])
            for g in range(16):
                pltpu.sync_copy(wbuf.at[pl.ds(128 * g, 128)],
                                deg_s.at[dix1.at[pl.ds(128 * g, 128)]],
                                add=True)
            return carry
        lax.fori_loop(0, PREP_CHUNKS, _chunk, None)

    plsc.subcore_barrier()

    @pl.when(s < 10)
    def _():
        off = pl.multiple_of(1000 * s, 8)
        pltpu.sync_copy(degm_s.at[pl.ds(off, 1000)], tmp1k)
        pltpu.sync_copy(tmp1k, degm_out.at[c, s, 0])
        pltpu.sync_copy(degd_s.at[pl.ds(off, 1000)], tmp1k)
        pltpu.sync_copy(tmp1k, degd_out.at[c, s, 0])


_edge_prep = pl.kernel(
    _edge_prep_body,
    out_type=(jax.ShapeDtypeStruct((E_PAD,), _f32),
              jax.ShapeDtypeStruct((E_PAD,), _f32),
              jax.ShapeDtypeStruct((NC, 10, 1, 1000), _f32),
              jax.ShapeDtypeStruct((NC, 10, 1, 1000), _f32)),
    mesh=_mesh,
    scratch_types=[
        pltpu.VMEM_SHARED((N + 8,), _f32),
        pltpu.VMEM_SHARED((N + 8,), _f32),
        pltpu.VMEM((PREP_CH,), _i32),
        pltpu.VMEM((PREP_CH,), _i32),
        pltpu.VMEM((PREP_CH,), _f32),
        pltpu.VMEM((1000,), _f32),
        pltpu.SemaphoreType.DMA,
        pltpu.SemaphoreType.DMA,
    ],
)


AGG_SCH = 1024


def _aggregate_body(fc, esplit, hm2, hd2, wm2, wd2, gm0, gm1, gd0, gd1,
                    dstm1, dstd1, zfc, aggm_out, aggd_out,
                    acc_s, gidx, dix1, wv, rows,
                    semi, semg0, semg1, sems0, sems1):
    c = lax.axis_index("c")
    s = lax.axis_index("s")
    nq = fc // 16
    per_t = E_PAD // (NC * NS) if esplit else E_PAD // NS
    nsch = per_t // AGG_SCH
    semg = (semg0, semg1)
    sems = (sems0, sems1)

    for (h2, w2, g0a, g1a, dst1, agg_out) in (
            (hm2, wm2, gm0, gm1, dstm1, aggm_out),
            (hd2, wd2, gd0, gd1, dstd1, aggd_out)):

        @pl.when(s < 10)
        def _():
            pltpu.sync_copy(zfc, rows.at[pl.ds(0, 200)])
            for t in range(5):
                off = pl.multiple_of(1000 * s + 200 * t, 8)
                pltpu.sync_copy(rows.at[pl.ds(0, 200)],
                                acc_s.at[pl.ds(off, 200)])
        plsc.subcore_barrier()

        tbase = s * per_t
        if esplit:
            tbase = tbase + c * (E_PAD // NC)

        def _load_idx(eoff):
            @pl.when(c == 0)
            def _():
                pltpu.async_copy(g0a.at[pl.ds(eoff, AGG_SCH)], gidx, semi)

            @pl.when(c == 1)
            def _():
                pltpu.async_copy(g1a.at[pl.ds(eoff, AGG_SCH)], gidx, semi)
            c2 = pltpu.async_copy(dst1.at[pl.ds(eoff, AGG_SCH)], dix1, semi)
            woff = pl.multiple_of(eoff // 16, 8)
            c3 = pltpu.async_copy(w2.at[pl.ds(woff, AGG_SCH // 16)], wv, semi)
            pltpu.make_async_copy(dst1.at[pl.ds(eoff, AGG_SCH)], gidx,
                                  semi).wait()
            c2.wait()
            c3.wait()

        def _issue_gather(gg, sl):
            pltpu.async_copy(h2.at[gidx.at[pl.ds(128 * gg, 128)]],
                             rows.at[pl.ds(128 * sl, 128)], semg[sl])

        def _wait_gather(gg, sl):
            pltpu.make_async_copy(h2.at[gidx.at[pl.ds(128 * gg, 128)]],
                                  rows.at[pl.ds(128 * sl, 128)],
                                  semg[sl]).wait()

        def _issue_scatter(gg, sl):
            pltpu.async_copy(rows.at[pl.ds(128 * sl, 128)],
                             acc_s.at[dix1.at[pl.ds(128 * gg, 128)]],
                             sems[sl], add=True)

        def _wait_scatter(gg, sl):
            pltpu.make_async_copy(rows.at[pl.ds(128 * sl, 128)],
                                  acc_s.at[dix1.at[pl.ds(128 * gg, 128)]],
                                  sems[sl]).wait()

        def _mul_group(gg, sl):
            @plsc.parallel_loop(0, 8, unroll=2)
            def _mul(jj):
                wg = wv[8 * gg + jj, pl.ds(0, 16)]
                for lane in range(16):
                    e = 128 * sl + jj * 16 + lane
                    ws = wg[lane]
                    for q in range(nq):
                        rows[e, pl.ds(16 * q, 16)] = (
                            rows[e, pl.ds(16 * q, 16)] * ws)

        _load_idx(tbase)
        _issue_gather(0, 0)
        _issue_gather(1, 1)

        def _chunk(k, cy):
            for gg in range(8):
                sl = gg & 1
                _wait_gather(gg, sl)
                _mul_group(gg, sl)
                _issue_scatter(gg, sl)
                if gg < 6:
                    _wait_scatter(gg, sl)
                    _issue_gather(gg + 2, sl)
            _wait_scatter(6, 0)
            _wait_scatter(7, 1)

            @pl.when(k + 1 < nsch)
            def _():
                _load_idx(tbase + (k + 1) * AGG_SCH)
                _issue_gather(0, 0)
                _issue_gather(1, 1)
            return cy
        lax.fori_loop(0, nsch, _chunk, None)
        plsc.subcore_barrier()

        @pl.when(s < 10)
        def _():
            for t in range(5):
                off = pl.multiple_of(1000 * s + 200 * t, 8)
                pltpu.sync_copy(acc_s.at[pl.ds(off, 200)],
                                rows.at[pl.ds(0, 200)])
                pltpu.sync_copy(rows.at[pl.ds(0, 200)],
                                agg_out.at[c].at[pl.ds(off, 200)])
        plsc.subcore_barrier()


def _make_aggregate(f, esplit):
    fc = f if esplit else f // 2
    return pl.kernel(
        functools.partial(_aggregate_body, fc, esplit),
        out_type=(jax.ShapeDtypeStruct((NC, N, fc), _f32),
                  jax.ShapeDtypeStruct((NC, N, fc), _f32)),
        mesh=_mesh,
        scratch_types=[
            pltpu.VMEM_SHARED((N + 1, fc), _f32),
            pltpu.VMEM((AGG_SCH,), _i32),
            pltpu.VMEM((AGG_SCH,), _i32),
            pltpu.VMEM((AGG_SCH // 16, 16), _f32),
            pltpu.VMEM((256, fc), _f32),
            pltpu.SemaphoreType.DMA,
            pltpu.SemaphoreType.DMA,
            pltpu.SemaphoreType.DMA,
            pltpu.SemaphoreType.DMA,
            pltpu.SemaphoreType.DMA,
        ],
    )


_aggregate_f1 = _make_aggregate(F1, False)
_aggregate_f2 = _make_aggregate(F2, True)


_RB = 1000


def _dinv_body(degm_ref, degd_ref, dm_ref, dd_ref):
    dm_ref[...] = lax.rsqrt(degm_ref[0, :] + degm_ref[1, :] + 1.0)
    dd_ref[...] = lax.rsqrt(degd_ref[0, :] + degd_ref[1, :] + 1.0)


_dinv_call = pl.pallas_call(
    _dinv_body,
    out_shape=(jax.ShapeDtypeStruct((N,), _f32),
               jax.ShapeDtypeStruct((N,), _f32)),
)


def _xw_body(x_ref, w_ref, dinv_ref, out_ref):
    out_ref[...] = dinv_ref[...] * jnp.dot(
        x_ref[...], w_ref[...], preferred_element_type=_f32)


def _xw_call(f_in, f_out):
    return pl.pallas_call(
        _xw_body,
        grid=(N // _RB,),
        in_specs=[
            pl.BlockSpec((_RB, f_in), lambda i: (i, 0)),
            pl.BlockSpec((f_in, f_out), lambda i: (0, 0)),
            pl.BlockSpec((_RB, 1), lambda i: (i, 0)),
        ],
        out_specs=pl.BlockSpec((_RB, f_out), lambda i: (i, 0)),
        out_shape=jax.ShapeDtypeStruct((N, f_out), _f32),
    )


def _layer2_body(agg_ref, ht_ref, dinv_ref, b1_ref, w2_ref, out_ref):
    aggf = jnp.concatenate([agg_ref[0], agg_ref[1]], axis=-1)
    z = jnp.maximum(dinv_ref[...] * (aggf + ht_ref[...]) + b1_ref[...], 0.0)
    out_ref[...] = dinv_ref[...] * jnp.dot(
        z, w2_ref[...], preferred_element_type=_f32)


_layer2_call = pl.pallas_call(
    _layer2_body,
    grid=(N // _RB,),
    in_specs=[
        pl.BlockSpec((NC, _RB, F1 // 2), lambda i: (0, i, 0)),
        pl.BlockSpec((_RB, F1), lambda i: (i, 0)),
        pl.BlockSpec((_RB, 1), lambda i: (i, 0)),
        pl.BlockSpec((1, F1), lambda i: (0, 0)),
        pl.BlockSpec((F1, F2), lambda i: (0, 0)),
    ],
    out_specs=pl.BlockSpec((_RB, F2), lambda i: (i, 0)),
    out_shape=jax.ShapeDtypeStruct((N, F2), _f32),
)


def _head_body(agg_ref, ht_ref, dinv_ref, b2_ref,
               wl1_ref, bl1_ref, wl2_ref, bl2_ref, wl3_ref, bl3_ref, out_ref):
    aggf = agg_ref[0] + agg_ref[1]
    xx = jnp.maximum(dinv_ref[...] * (aggf + ht_ref[...]) + b2_ref[...], 0.0)
    x1 = jnp.maximum(jnp.dot(xx, wl1_ref[...], preferred_element_type=_f32)
                     + bl1_ref[...], 0.0)
    x2 = jnp.maximum(jnp.dot(x1, wl2_ref[...], preferred_element_type=_f32)
                     + bl2_ref[...], 0.0)
    out_ref[...] = jnp.maximum(
        jnp.dot(x2, wl3_ref[...], preferred_element_type=_f32)
        + bl3_ref[...], 0.0)


_head_call = pl.pallas_call(
    _head_body,
    grid=(N // _RB,),
    in_specs=[
        pl.BlockSpec((NC, _RB, F2), lambda i: (0, i, 0)),
        pl.BlockSpec((_RB, F2), lambda i: (i, 0)),
        pl.BlockSpec((_RB, 1), lambda i: (i, 0)),
        pl.BlockSpec((1, F2), lambda i: (0, 0)),
        pl.BlockSpec((F2, 256), lambda i: (0, 0)),
        pl.BlockSpec((1, 256), lambda i: (0, 0)),
        pl.BlockSpec((256, 128), lambda i: (0, 0)),
        pl.BlockSpec((1, 128), lambda i: (0, 0)),
        pl.BlockSpec((128, KOUT), lambda i: (0, 0)),
        pl.BlockSpec((1, KOUT), lambda i: (0, 0)),
    ],
    out_specs=pl.BlockSpec((_RB, KOUT), lambda i: (i, 0)),
    out_shape=jax.ShapeDtypeStruct((N, KOUT), _f32),
)

_FRB = 400


def _final_body(xf_ref, yf_ref, out_ref):
    out_ref[...] = lax.dot_general(
        xf_ref[...], yf_ref[...], (((1,), (1,)), ((), ())),
        preferred_element_type=_f32)


_final_call = pl.pallas_call(
    _final_body,
    grid=(N // _FRB,),
    in_specs=[
        pl.BlockSpec((_FRB, KOUT), lambda i: (i, 0)),
        pl.BlockSpec((N, KOUT), lambda i: (0, 0)),
    ],
    out_specs=pl.BlockSpec((_FRB, N), lambda i: (i, 0)),
    out_shape=jax.ShapeDtypeStruct((N, N), _f32),
)


def _pad_edges(ei):
    ei = ei.astype(_i32)
    src = jnp.concatenate([ei[0], jnp.zeros((E_PAD - E,), _i32)])
    dst = jnp.concatenate([ei[1], jnp.full((E_PAD - E,), N, _i32)])
    return src, dst


def kernel(mm_data, dd_data, x_m, x_d, Wgx1, bgx1, Wgx2, bgx2, Wgy1, bgy1,
           Wgy2, bgy2, Wlx1, blx1, Wlx2, blx2, Wlx3, blx3, Wly1, bly1,
           Wly2, bly2, Wly3, bly3, mm_edge_index, dd_edge_index):
    srcm1, dstm1 = _pad_edges(mm_edge_index)
    srcd1, dstd1 = _pad_edges(dd_edge_index)
    fidxm = srcm1 * N + dstm1
    fidxd = srcd1 * N + dstd1
    gm0 = srcm1 * 2
    gm1 = gm0 + 1
    gd0 = srcd1 * 2
    gd1 = gd0 + 1
    datamf = mm_data.reshape(N * N)
    datadf = dd_data.reshape(N * N)
    z1k = jnp.zeros((1000,), _f32)
    z100 = jnp.zeros((200, F1 // 2), _f32)

    wm, wd, degm, degd = _edge_prep(datamf, datadf, fidxm, fidxd,
                                    dstm1, dstd1, z1k)
    wm2 = wm.reshape(E_PAD // 16, 16)
    wd2 = wd.reshape(E_PAD // 16, 16)

    dinv_m, dinv_d = _dinv_call(degm.reshape(NC, N), degd.reshape(NC, N))
    dm2 = dinv_m.reshape(N, 1)
    dd2 = dinv_d.reshape(N, 1)

    xw = _xw_call(FIN, F1)
    htm1 = xw(x_m, Wgx1, dm2)
    htd1 = xw(x_d, Wgy1, dd2)

    aggm1, aggd1 = _aggregate_f1(htm1.reshape(2 * N, F1 // 2),
                                 htd1.reshape(2 * N, F1 // 2),
                                 wm2, wd2, gm0, gm1, gd0, gd1,
                                 dstm1, dstd1, z100)

    b1m = bgx1.reshape(1, F1)
    b1d = bgy1.reshape(1, F1)
    htm2 = _layer2_call(aggm1, htm1, dm2, b1m, Wgx2)
    htd2 = _layer2_call(aggd1, htd1, dd2, b1d, Wgy2)

    aggm2, aggd2 = _aggregate_f2(htm2, htd2,
                                 wm2, wd2, srcm1, srcm1, srcd1, srcd1,
                                 dstm1, dstd1, z100)

    xf = _head_call(aggm2, htm2, dm2, bgx2.reshape(1, F2),
                    Wlx1, blx1.reshape(1, 256), Wlx2, blx2.reshape(1, 128),
                    Wlx3, blx3.reshape(1, KOUT))
    yf = _head_call(aggd2, htd2, dd2, bgy2.reshape(1, F2),
                    Wly1, bly1.reshape(1, 256), Wly2, bly2.reshape(1, 128),
                    Wly3, bly3.reshape(1, KOUT))

    return _final_call(xf, yf)


# speedup vs baseline: 5.6748x; 1.1206x over previous
import functools

import jax
import jax.numpy as jnp
from jax import lax
from jax.experimental import pallas as pl
from jax.experimental.pallas import tpu as pltpu
from jax.experimental.pallas import tpu_sc as plsc

N = 10000
FIN = 128
F1 = 256
F2 = 128
KOUT = 64
E = 320000
E_PAD = 327680
NC = 2
NS = 16

_mesh = plsc.VectorSubcoreMesh(
    core_axis_name="c", subcore_axis_name="s", num_cores=NC, num_subcores=NS)

_f32 = jnp.float32
_i32 = jnp.int32


PREP_CH = 2048
PREP_PER_W = E_PAD // (NC * NS)
PREP_CHUNKS = PREP_PER_W // PREP_CH


def _edge_prep_body(datam, datad, fidxm, fidxd, dstm1, dstd1, z1k,
                    wm_out, wd_out, degm_out, degd_out,
                    degm_s, degd_s,
                    ridx, dix1, wbuf, tmp1k, sem, semi):
    c = lax.axis_index("c")
    s = lax.axis_index("s")
    wid = s * NC + c
    base = wid * PREP_PER_W

    @pl.when(s == 0)
    def _():
        pltpu.sync_copy(z1k, tmp1k)
        for t in range(10):
            pltpu.sync_copy(tmp1k, degm_s.at[pl.ds(1000 * t, 1000)])
            pltpu.sync_copy(tmp1k, degd_s.at[pl.ds(1000 * t, 1000)])
    plsc.subcore_barrier()

    for (dataf, fidx, dst1, w_out, deg_s) in (
            (datam, fidxm, dstm1, wm_out, degm_s),
            (datad, fidxd, dstd1, wd_out, degd_s)):

        def _chunk(k, carry):
            eoff = base + k * PREP_CH
            c1 = pltpu.async_copy(fidx.at[pl.ds(eoff, PREP_CH)], ridx, semi)
            c2 = pltpu.async_copy(dst1.at[pl.ds(eoff, PREP_CH)], dix1, semi)
            c1.wait()
            c2.wait()

            cps = [pltpu.async_copy(dataf.at[ridx.at[pl.ds(128 * g, 128)]],
                                    wbuf.at[pl.ds(128 * g, 128)], sem)
                   for g in range(16)]
            for cp in cps:
                cp.wait()

            pltpu.sync_copy(wbuf, w_out.at[pl.ds(eoff, PREP_CH)])
            for g in range(16):
                pltpu.sync_copy(wbuf.at[pl.ds(128 * g, 128)],
                                deg_s.at[dix1.at[pl.ds(128 * g, 128)]],
                                add=True)
            return carry
        lax.fori_loop(0, PREP_CHUNKS, _chunk, None)

    plsc.subcore_barrier()

    @pl.when(s < 10)
    def _():
        off = pl.multiple_of(1000 * s, 8)
        pltpu.sync_copy(degm_s.at[pl.ds(off, 1000)], tmp1k)
        pltpu.sync_copy(tmp1k, degm_out.at[c, s, 0])
        pltpu.sync_copy(degd_s.at[pl.ds(off, 1000)], tmp1k)
        pltpu.sync_copy(tmp1k, degd_out.at[c, s, 0])


_edge_prep = pl.kernel(
    _edge_prep_body,
    out_type=(jax.ShapeDtypeStruct((E_PAD,), _f32),
              jax.ShapeDtypeStruct((E_PAD,), _f32),
              jax.ShapeDtypeStruct((NC, 10, 1, 1000), _f32),
              jax.ShapeDtypeStruct((NC, 10, 1, 1000), _f32)),
    mesh=_mesh,
    scratch_types=[
        pltpu.VMEM_SHARED((N + 8,), _f32),
        pltpu.VMEM_SHARED((N + 8,), _f32),
        pltpu.VMEM((PREP_CH,), _i32),
        pltpu.VMEM((PREP_CH,), _i32),
        pltpu.VMEM((PREP_CH,), _f32),
        pltpu.VMEM((1000,), _f32),
        pltpu.SemaphoreType.DMA,
        pltpu.SemaphoreType.DMA,
    ],
)


AGG_SCH = 1024


def _aggregate_body(fc, esplit, hm2, hd2, wm2, wd2, gm0, gm1, gd0, gd1,
                    dstm1, dstd1, zfc, aggm_out, aggd_out,
                    acc_s, gidx, dix1, wv, rows,
                    semi, semg0, semg1, sems0, sems1):
    c = lax.axis_index("c")
    s = lax.axis_index("s")
    nq = fc // 16
    per_t = E_PAD // (NC * NS) if esplit else E_PAD // NS
    nsch = per_t // AGG_SCH
    semg = (semg0, semg1)
    sems = (sems0, sems1)

    for (h2, w2, g0a, g1a, dst1, agg_out) in (
            (hm2, wm2, gm0, gm1, dstm1, aggm_out),
            (hd2, wd2, gd0, gd1, dstd1, aggd_out)):

        @pl.when(s < 10)
        def _():
            pltpu.sync_copy(zfc, rows.at[pl.ds(0, 200)])
            for t in range(5):
                off = pl.multiple_of(1000 * s + 200 * t, 8)
                pltpu.sync_copy(rows.at[pl.ds(0, 200)],
                                acc_s.at[pl.ds(off, 200)])
        plsc.subcore_barrier()

        tbase = s * per_t
        if esplit:
            tbase = tbase + c * (E_PAD // NC)

        def _load_idx(eoff):
            @pl.when(c == 0)
            def _():
                pltpu.async_copy(g0a.at[pl.ds(eoff, AGG_SCH)], gidx, semi)

            @pl.when(c == 1)
            def _():
                pltpu.async_copy(g1a.at[pl.ds(eoff, AGG_SCH)], gidx, semi)
            c2 = pltpu.async_copy(dst1.at[pl.ds(eoff, AGG_SCH)], dix1, semi)
            woff = pl.multiple_of(eoff // 16, 8)
            c3 = pltpu.async_copy(w2.at[pl.ds(woff, AGG_SCH // 16)], wv, semi)
            pltpu.make_async_copy(dst1.at[pl.ds(eoff, AGG_SCH)], gidx,
                                  semi).wait()
            c2.wait()
            c3.wait()

        def _issue_gather(gg, sl):
            pltpu.async_copy(h2.at[gidx.at[pl.ds(128 * gg, 128)]],
                             rows.at[pl.ds(128 * sl, 128)], semg[sl])

        def _wait_gather(gg, sl):
            pltpu.make_async_copy(h2.at[gidx.at[pl.ds(128 * gg, 128)]],
                                  rows.at[pl.ds(128 * sl, 128)],
                                  semg[sl]).wait()

        def _issue_scatter(gg, sl):
            pltpu.async_copy(rows.at[pl.ds(128 * sl, 128)],
                             acc_s.at[dix1.at[pl.ds(128 * gg, 128)]],
                             sems[sl], add=True)

        def _wait_scatter(gg, sl):
            pltpu.make_async_copy(rows.at[pl.ds(128 * sl, 128)],
                                  acc_s.at[dix1.at[pl.ds(128 * gg, 128)]],
                                  sems[sl]).wait()

        def _mul_group(gg, sl):
            @plsc.parallel_loop(0, 8, unroll=2)
            def _mul(jj):
                wg = wv[8 * gg + jj, pl.ds(0, 16)]
                for lane in range(16):
                    e = 128 * sl + jj * 16 + lane
                    ws = wg[lane]
                    for q in range(nq):
                        rows[e, pl.ds(16 * q, 16)] = (
                            rows[e, pl.ds(16 * q, 16)] * ws)

        _load_idx(tbase)
        _issue_gather(0, 0)
        _issue_gather(1, 1)

        def _chunk(k, cy):
            for gg in range(8):
                sl = gg & 1
                _wait_gather(gg, sl)
                _mul_group(gg, sl)
                _issue_scatter(gg, sl)
                if gg < 6:
                    _wait_scatter(gg, sl)
                    _issue_gather(gg + 2, sl)
            _wait_scatter(6, 0)
            _wait_scatter(7, 1)

            @pl.when(k + 1 < nsch)
            def _():
                _load_idx(tbase + (k + 1) * AGG_SCH)
                _issue_gather(0, 0)
                _issue_gather(1, 1)
            return cy
        lax.fori_loop(0, nsch, _chunk, None)
        plsc.subcore_barrier()

        @pl.when(s < 10)
        def _():
            for t in range(5):
                off = pl.multiple_of(1000 * s + 200 * t, 8)
                pltpu.sync_copy(acc_s.at[pl.ds(off, 200)],
                                rows.at[pl.ds(0, 200)])
                pltpu.sync_copy(rows.at[pl.ds(0, 200)],
                                agg_out.at[c].at[pl.ds(off, 200)])
        plsc.subcore_barrier()


def _make_aggregate(f, esplit):
    fc = f if esplit else f // 2
    return pl.kernel(
        functools.partial(_aggregate_body, fc, esplit),
        out_type=(jax.ShapeDtypeStruct((NC, N, fc), _f32),
                  jax.ShapeDtypeStruct((NC, N, fc), _f32)),
        mesh=_mesh,
        scratch_types=[
            pltpu.VMEM_SHARED((N + 1, fc), _f32),
            pltpu.VMEM((AGG_SCH,), _i32),
            pltpu.VMEM((AGG_SCH,), _i32),
            pltpu.VMEM((AGG_SCH // 16, 16), _f32),
            pltpu.VMEM((256, fc), _f32),
            pltpu.SemaphoreType.DMA,
            pltpu.SemaphoreType.DMA,
            pltpu.SemaphoreType.DMA,
            pltpu.SemaphoreType.DMA,
            pltpu.SemaphoreType.DMA,
        ],
    )


_aggregate = _make_aggregate(F2, True)


_RB = 1000


def _dinv_body(degm_ref, degd_ref, dm_ref, dd_ref):
    dm_ref[...] = lax.rsqrt(degm_ref[0, :] + degm_ref[1, :] + 1.0)
    dd_ref[...] = lax.rsqrt(degd_ref[0, :] + degd_ref[1, :] + 1.0)


_dinv_call = pl.pallas_call(
    _dinv_body,
    out_shape=(jax.ShapeDtypeStruct((N,), _f32),
               jax.ShapeDtypeStruct((N,), _f32)),
)


def _scale_body(x_ref, dinv_ref, out_ref):
    out_ref[...] = dinv_ref[...] * x_ref[...]


_scale_call = pl.pallas_call(
    _scale_body,
    grid=(N // _RB,),
    in_specs=[
        pl.BlockSpec((_RB, FIN), lambda i: (i, 0)),
        pl.BlockSpec((_RB, 1), lambda i: (i, 0)),
    ],
    out_specs=pl.BlockSpec((_RB, FIN), lambda i: (i, 0)),
    out_shape=jax.ShapeDtypeStruct((N, FIN), _f32),
)


def _mid_body(agg_ref, ht0_ref, dinv_ref, b1_ref, w1_ref, w2_ref, out_ref):
    t = agg_ref[0] + agg_ref[1] + ht0_ref[...]
    z1 = jnp.maximum(
        dinv_ref[...] * jnp.dot(t, w1_ref[...], preferred_element_type=_f32)
        + b1_ref[...], 0.0)
    out_ref[...] = dinv_ref[...] * jnp.dot(
        z1, w2_ref[...], preferred_element_type=_f32)


_mid_call = pl.pallas_call(
    _mid_body,
    grid=(N // _RB,),
    in_specs=[
        pl.BlockSpec((NC, _RB, FIN), lambda i: (0, i, 0)),
        pl.BlockSpec((_RB, FIN), lambda i: (i, 0)),
        pl.BlockSpec((_RB, 1), lambda i: (i, 0)),
        pl.BlockSpec((1, F1), lambda i: (0, 0)),
        pl.BlockSpec((FIN, F1), lambda i: (0, 0)),
        pl.BlockSpec((F1, F2), lambda i: (0, 0)),
    ],
    out_specs=pl.BlockSpec((_RB, F2), lambda i: (i, 0)),
    out_shape=jax.ShapeDtypeStruct((N, F2), _f32),
)


def _head_body(agg_ref, ht_ref, dinv_ref, b2_ref,
               wl1_ref, bl1_ref, wl2_ref, bl2_ref, wl3_ref, bl3_ref, out_ref):
    aggf = agg_ref[0] + agg_ref[1]
    xx = jnp.maximum(dinv_ref[...] * (aggf + ht_ref[...]) + b2_ref[...], 0.0)
    x1 = jnp.maximum(jnp.dot(xx, wl1_ref[...], preferred_element_type=_f32)
                     + bl1_ref[...], 0.0)
    x2 = jnp.maximum(jnp.dot(x1, wl2_ref[...], preferred_element_type=_f32)
                     + bl2_ref[...], 0.0)
    out_ref[...] = jnp.maximum(
        jnp.dot(x2, wl3_ref[...], preferred_element_type=_f32)
        + bl3_ref[...], 0.0)


_head_call = pl.pallas_call(
    _head_body,
    grid=(N // _RB,),
    in_specs=[
        pl.BlockSpec((NC, _RB, F2), lambda i: (0, i, 0)),
        pl.BlockSpec((_RB, F2), lambda i: (i, 0)),
        pl.BlockSpec((_RB, 1), lambda i: (i, 0)),
        pl.BlockSpec((1, F2), lambda i: (0, 0)),
        pl.BlockSpec((F2, 256), lambda i: (0, 0)),
        pl.BlockSpec((1, 256), lambda i: (0, 0)),
        pl.BlockSpec((256, 128), lambda i: (0, 0)),
        pl.BlockSpec((1, 128), lambda i: (0, 0)),
        pl.BlockSpec((128, KOUT), lambda i: (0, 0)),
        pl.BlockSpec((1, KOUT), lambda i: (0, 0)),
    ],
    out_specs=pl.BlockSpec((_RB, KOUT), lambda i: (i, 0)),
    out_shape=jax.ShapeDtypeStruct((N, KOUT), _f32),
)

_FRB = 400


def _final_body(xf_ref, yf_ref, out_ref):
    out_ref[...] = lax.dot_general(
        xf_ref[...], yf_ref[...], (((1,), (1,)), ((), ())),
        preferred_element_type=_f32)


_final_call = pl.pallas_call(
    _final_body,
    grid=(N // _FRB,),
    in_specs=[
        pl.BlockSpec((_FRB, KOUT), lambda i: (i, 0)),
        pl.BlockSpec((N, KOUT), lambda i: (0, 0)),
    ],
    out_specs=pl.BlockSpec((_FRB, N), lambda i: (i, 0)),
    out_shape=jax.ShapeDtypeStruct((N, N), _f32),
)


def _pad_edges(ei):
    ei = ei.astype(_i32)
    src = jnp.concatenate([ei[0], jnp.zeros((E_PAD - E,), _i32)])
    dst = jnp.concatenate([ei[1], jnp.full((E_PAD - E,), N, _i32)])
    return src, dst


def kernel(mm_data, dd_data, x_m, x_d, Wgx1, bgx1, Wgx2, bgx2, Wgy1, bgy1,
           Wgy2, bgy2, Wlx1, blx1, Wlx2, blx2, Wlx3, blx3, Wly1, bly1,
           Wly2, bly2, Wly3, bly3, mm_edge_index, dd_edge_index):
    srcm1, dstm1 = _pad_edges(mm_edge_index)
    srcd1, dstd1 = _pad_edges(dd_edge_index)
    fidxm = srcm1 * N + dstm1
    fidxd = srcd1 * N + dstd1
    datamf = mm_data.reshape(N * N)
    datadf = dd_data.reshape(N * N)
    z1k = jnp.zeros((1000,), _f32)
    z200 = jnp.zeros((200, F2), _f32)

    wm, wd, degm, degd = _edge_prep(datamf, datadf, fidxm, fidxd,
                                    dstm1, dstd1, z1k)
    wm2 = wm.reshape(E_PAD // 16, 16)
    wd2 = wd.reshape(E_PAD // 16, 16)

    dinv_m, dinv_d = _dinv_call(degm.reshape(NC, N), degd.reshape(NC, N))
    dm2 = dinv_m.reshape(N, 1)
    dd2 = dinv_d.reshape(N, 1)

    ht0m = _scale_call(x_m, dm2)
    ht0d = _scale_call(x_d, dd2)

    aggm1, aggd1 = _aggregate(ht0m, ht0d, wm2, wd2,
                              srcm1, srcm1, srcd1, srcd1,
                              dstm1, dstd1, z200)

    htm2 = _mid_call(aggm1, ht0m, dm2, bgx1.reshape(1, F1), Wgx1, Wgx2)
    htd2 = _mid_call(aggd1, ht0d, dd2, bgy1.reshape(1, F1), Wgy1, Wgy2)

    aggm2, aggd2 = _aggregate(htm2, htd2, wm2, wd2,
                              srcm1, srcm1, srcd1, srcd1,
                              dstm1, dstd1, z200)

    xf = _head_call(aggm2, htm2, dm2, bgx2.reshape(1, F2),
                    Wlx1, blx1.reshape(1, 256), Wlx2, blx2.reshape(1, 128),
                    Wlx3, blx3.reshape(1, KOUT))
    yf = _head_call(aggd2, htd2, dd2, bgy2.reshape(1, F2),
                    Wly1, bly1.reshape(1, 256), Wly2, bly2.reshape(1, 128),
                    Wly3, bly3.reshape(1, KOUT))

    return _final_call(xf, yf)
